# baseline (speedup 1.0000x reference)
import functools

import jax
import jax.numpy as jnp
from jax import lax
from jax.experimental import pallas as pl
from jax.experimental.pallas import tpu as pltpu
from jax.experimental.pallas import tpu_sc as plsc

M = 40000
N = 32
NUM_BINS = 64
Z_MIN = -3.0
INV_BIN_SIZE = 16.0
FEAT = 64

NUM_WORKERS = 32
LANES = 16
GROUPS_PER_CHUNK = 8
CHUNK = LANES * GROUPS_PER_CHUNK
MP = -(-M // CHUNK) * CHUNK
NUM_CHUNKS = MP // CHUNK
CHUNKS_PER_WORKER = -(-NUM_CHUNKS // NUM_WORKERS)


def _sc_hist_body(zt_hbm, rt_hbm, np_hbm, hist_hbm,
                  z_v, r_v, np_v, h_v,
                  isem0, isem1, isem2, osem0, osem1, osem2):
    wid = lax.axis_index("s") * 2 + lax.axis_index("c")
    lanes = lax.iota(jnp.int32, 16)
    ones = jnp.ones((16,), jnp.float32)
    zeros = jnp.zeros((16,), jnp.float32)
    isems = (isem0, isem1, isem2)
    osems = (osem0, osem1, osem2)

    def chunk_index(i):
        return i * NUM_WORKERS + wid

    def start_in(i, b):
        m0 = chunk_index(i) * CHUNK
        pltpu.make_async_copy(zt_hbm.at[:, pl.ds(m0, CHUNK)],
                              z_v.at[b], isems[b]).start()
        pltpu.make_async_copy(rt_hbm.at[:, pl.ds(m0, CHUNK)],
                              r_v.at[b], isems[b]).start()
        pltpu.make_async_copy(np_hbm.at[pl.ds(m0, CHUNK)],
                              np_v.at[b], isems[b]).start()

    def wait_in(b):
        pltpu.make_async_copy(zt_hbm.at[:, pl.ds(0, CHUNK)],
                              z_v.at[b], isems[b]).wait()
        pltpu.make_async_copy(rt_hbm.at[:, pl.ds(0, CHUNK)],
                              r_v.at[b], isems[b]).wait()
        pltpu.make_async_copy(np_hbm.at[pl.ds(0, CHUNK)],
                              np_v.at[b], isems[b]).wait()

    def start_out(i, b):
        m0 = chunk_index(i) * CHUNK
        pltpu.make_async_copy(h_v.at[b], hist_hbm.at[:, pl.ds(m0, CHUNK)],
                              osems[b]).start()

    def wait_out(b):
        pltpu.make_async_copy(h_v.at[b], hist_hbm.at[:, pl.ds(0, CHUNK)],
                              osems[b]).wait()

    def compute(i, b):
        def zero_body(bi, zcarry):
            for k in range(4):
                for j in range(CHUNK // 16):
                    h_v[b, bi * 4 + k, pl.ds(j * 16, 16)] = zeros
            return zcarry

        lax.fori_loop(0, 3 * NUM_BINS // 4, zero_body, 0)
        wait_in(b)

        def group_body(l, gcarry):
            off = l * 16
            npk = np_v[b, pl.ds(off, 16)]
            pil = lanes + off
            for p in range(N):
                z = z_v[b, p, pl.ds(off, 16)]
                r = r_v[b, p, pl.ds(off, 16)]
                t = (z - Z_MIN) * INV_BIN_SIZE
                bin_ = t.astype(jnp.int32)
                bin_ = jnp.minimum(jnp.maximum(bin_, 0), NUM_BINS - 1)
                valid = jnp.full((16,), p, jnp.int32) < npk
                plsc.addupdate_scatter(h_v.at[b], [bin_, pil], ones,
                                       mask=valid)
                plsc.addupdate_scatter(h_v.at[b], [bin_ + NUM_BINS, pil], z,
                                       mask=valid)
                plsc.addupdate_scatter(h_v.at[b], [bin_ + 2 * NUM_BINS, pil],
                                       r, mask=valid)
            return gcarry

        lax.fori_loop(0, GROUPS_PER_CHUNK, group_body, 0)
        start_out(i, b)

    start_in(0, 0)
    start_in(1, 1)
    for i in range(CHUNKS_PER_WORKER):
        b = i % 3
        nxt = i + 2
        if nxt < CHUNKS_PER_WORKER:
            if nxt <= 8:
                start_in(nxt, nxt % 3)
            else:
                @pl.when(chunk_index(nxt) < NUM_CHUNKS)
                def _(nxt=nxt, nb=nxt % 3):
                    start_in(nxt, nb)
        if i >= 3:
            wait_out(b)
        if i <= 8:
            compute(i, b)
        else:
            @pl.when(chunk_index(i) < NUM_CHUNKS)
            def _(i=i, b=b):
                compute(i, b)
    wait_out(7 % 3)
    wait_out(8 % 3)

    @pl.when(chunk_index(CHUNKS_PER_WORKER - 1) < NUM_CHUNKS)
    def _():
        wait_out(9 % 3)


_sc_hist = functools.partial(
    pl.kernel,
    out_type=jax.ShapeDtypeStruct((3 * NUM_BINS, MP), jnp.float32),
    mesh=plsc.VectorSubcoreMesh(core_axis_name="c", subcore_axis_name="s"),
    compiler_params=pltpu.CompilerParams(needs_layout_passes=False),
    scratch_types=[
        pltpu.VMEM((3, N, CHUNK), jnp.float32),
        pltpu.VMEM((3, N, CHUNK), jnp.float32),
        pltpu.VMEM((3, CHUNK), jnp.int32),
        pltpu.VMEM((3, 3 * NUM_BINS, CHUNK), jnp.float32),
        pltpu.SemaphoreType.DMA,
        pltpu.SemaphoreType.DMA,
        pltpu.SemaphoreType.DMA,
        pltpu.SemaphoreType.DMA,
        pltpu.SemaphoreType.DMA,
        pltpu.SemaphoreType.DMA,
    ],
)(_sc_hist_body)


BM = 16384
G1 = pl.cdiv(MP, BM)


def _tc1_body(h_ref, wc_ref, wz_ref, wr_ref, b_ref,
              x_ref, st_ref, acc):
    i = pl.program_id(0)

    @pl.when(i == 0)
    def _():
        acc[...] = jnp.zeros_like(acc)

    cnt = h_ref[0:NUM_BINS, :]
    denom = cnt + 1e-5
    mz = h_ref[NUM_BINS:2 * NUM_BINS, :] / denom
    mr = h_ref[2 * NUM_BINS:3 * NUM_BINS, :] / denom
    x = (jnp.dot(wc_ref[...], cnt, preferred_element_type=jnp.float32)
         + jnp.dot(wz_ref[...], mz, preferred_element_type=jnp.float32)
         + jnp.dot(wr_ref[...], mr, preferred_element_type=jnp.float32)
         + b_ref[...])
    x_ref[...] = x
    mcol = i * BM + lax.broadcasted_iota(jnp.int32, (FEAT, BM), 1)
    xm = jnp.where(mcol < M, x, 0.0)
    acc[:, 0:1] += jnp.sum(xm, axis=1, keepdims=True)
    acc[:, 1:2] += jnp.sum(xm * xm, axis=1, keepdims=True)

    @pl.when(i == G1 - 1)
    def _():
        st_ref[...] = acc[...]


def _tc2_body(x_ref, st_ref, g_ref, be_ref, o_ref):
    s = st_ref[:, 0:1]
    ss = st_ref[:, 1:2]
    mu = s * (1.0 / M)
    var = jnp.maximum(ss * (1.0 / M) - mu * mu, 0.0)
    inv = 1.0 / jnp.sqrt(var + 1e-5)
    scale = g_ref[...] * inv
    shift = be_ref[...] - mu * scale
    o_ref[...] = jnp.maximum(x_ref[...] * scale + shift, 0.0)


def kernel(features, num_points, coors, W, b, gamma, beta):
    del coors
    zt = jnp.pad(features[:, :, 2].T, ((0, 0), (0, MP - M)))
    rt = jnp.pad(features[:, :, 3].T, ((0, 0), (0, MP - M)))
    np32 = jnp.pad(num_points.astype(jnp.int32), (0, MP - M))

    hist = _sc_hist(zt, rt, np32)

    wc = jnp.asarray(W[:, 0::3])
    wz = jnp.asarray(W[:, 1::3])
    wr = jnp.asarray(W[:, 2::3])

    xt, st = pl.pallas_call(
        _tc1_body,
        grid=(G1,),
        in_specs=[
            pl.BlockSpec((3 * NUM_BINS, BM), lambda i: (0, i)),
            pl.BlockSpec((FEAT, NUM_BINS), lambda i: (0, 0)),
            pl.BlockSpec((FEAT, NUM_BINS), lambda i: (0, 0)),
            pl.BlockSpec((FEAT, NUM_BINS), lambda i: (0, 0)),
            pl.BlockSpec((FEAT, 1), lambda i: (0, 0)),
        ],
        out_specs=[
            pl.BlockSpec((FEAT, BM), lambda i: (0, i)),
            pl.BlockSpec((FEAT, 128), lambda i: (0, 0)),
        ],
        out_shape=[
            jax.ShapeDtypeStruct((FEAT, M), jnp.float32),
            jax.ShapeDtypeStruct((FEAT, 128), jnp.float32),
        ],
        scratch_shapes=[pltpu.VMEM((FEAT, 128), jnp.float32)],
    )(hist, wc, wz, wr, b.reshape(FEAT, 1))

    out_t = pl.pallas_call(
        _tc2_body,
        grid=(G1,),
        in_specs=[
            pl.BlockSpec((FEAT, BM), lambda i: (0, i)),
            pl.BlockSpec((FEAT, 128), lambda i: (0, 0)),
            pl.BlockSpec((FEAT, 1), lambda i: (0, 0)),
            pl.BlockSpec((FEAT, 1), lambda i: (0, 0)),
        ],
        out_specs=pl.BlockSpec((FEAT, BM), lambda i: (0, i)),
        out_shape=jax.ShapeDtypeStruct((FEAT, M), jnp.float32),
    )(xt, st, gamma.reshape(FEAT, 1), beta.reshape(FEAT, 1))

    return out_t.T


# speedup vs baseline: 1.0039x; 1.0039x over previous
import functools

import jax
import jax.numpy as jnp
from jax import lax
from jax.experimental import pallas as pl
from jax.experimental.pallas import tpu as pltpu
from jax.experimental.pallas import tpu_sc as plsc

M = 40000
N = 32
NUM_BINS = 64
Z_MIN = -3.0
INV_BIN_SIZE = 16.0
FEAT = 64

NUM_WORKERS = 32
LANES = 16
GROUPS_PER_CHUNK = 8
CHUNK = LANES * GROUPS_PER_CHUNK
MP = -(-M // CHUNK) * CHUNK
NUM_CHUNKS = MP // CHUNK
CHUNKS_PER_WORKER = -(-NUM_CHUNKS // NUM_WORKERS)


def _sc_hist_body(zt_hbm, rt_hbm, np_hbm, hist_hbm,
                  z_v, r_v, np_v, h_v,
                  isem0, isem1, osem0, osem1):
    wid = lax.axis_index("s") * 2 + lax.axis_index("c")
    lanes = lax.iota(jnp.int32, 16)
    ones = jnp.ones((16,), jnp.float32)
    zeros = jnp.zeros((16,), jnp.float32)
    isems = (isem0, isem1)
    osems = (osem0, osem1)

    def chunk_index(i):
        return i * NUM_WORKERS + wid

    def start_in(i, b):
        m0 = chunk_index(i) * CHUNK
        pltpu.make_async_copy(zt_hbm.at[:, pl.ds(m0, CHUNK)],
                              z_v.at[b], isems[b]).start()
        pltpu.make_async_copy(rt_hbm.at[:, pl.ds(m0, CHUNK)],
                              r_v.at[b], isems[b]).start()
        pltpu.make_async_copy(np_hbm.at[pl.ds(m0, CHUNK)],
                              np_v.at[b], isems[b]).start()

    def wait_in(b):
        pltpu.make_async_copy(zt_hbm.at[:, pl.ds(0, CHUNK)],
                              z_v.at[b], isems[b]).wait()
        pltpu.make_async_copy(rt_hbm.at[:, pl.ds(0, CHUNK)],
                              r_v.at[b], isems[b]).wait()
        pltpu.make_async_copy(np_hbm.at[pl.ds(0, CHUNK)],
                              np_v.at[b], isems[b]).wait()

    def start_out(i, b):
        m0 = chunk_index(i) * CHUNK
        pltpu.make_async_copy(h_v.at[b], hist_hbm.at[:, pl.ds(m0, CHUNK)],
                              osems[b]).start()

    def wait_out(b):
        pltpu.make_async_copy(h_v.at[b], hist_hbm.at[:, pl.ds(0, CHUNK)],
                              osems[b]).wait()

    def compute(i, b):
        def zero_body(bi, zcarry):
            for k in range(4):
                for j in range(CHUNK // 16):
                    h_v[b, bi * 4 + k, pl.ds(j * 16, 16)] = zeros
            return zcarry

        lax.fori_loop(0, 3 * NUM_BINS // 4, zero_body, 0)
        wait_in(b)

        def group_body(l, gcarry):
            off = l * 16
            npk = np_v[b, pl.ds(off, 16)]
            pil = lanes + off
            for p in range(N):
                z = z_v[b, p, pl.ds(off, 16)]
                r = r_v[b, p, pl.ds(off, 16)]
                t = (z - Z_MIN) * INV_BIN_SIZE
                bin_ = t.astype(jnp.int32)
                bin_ = jnp.minimum(jnp.maximum(bin_, 0), NUM_BINS - 1)
                valid = jnp.full((16,), p, jnp.int32) < npk
                plsc.addupdate_scatter(h_v.at[b], [bin_, pil], ones,
                                       mask=valid)
                plsc.addupdate_scatter(h_v.at[b], [bin_ + NUM_BINS, pil], z,
                                       mask=valid)
                plsc.addupdate_scatter(h_v.at[b], [bin_ + 2 * NUM_BINS, pil],
                                       r, mask=valid)
            return gcarry

        lax.fori_loop(0, GROUPS_PER_CHUNK, group_body, 0)
        start_out(i, b)

    start_in(0, 0)
    for i in range(CHUNKS_PER_WORKER):
        b = i % 2
        nxt = i + 1
        if nxt < CHUNKS_PER_WORKER:
            if nxt <= 8:
                start_in(nxt, 1 - b)
            else:
                @pl.when(chunk_index(nxt) < NUM_CHUNKS)
                def _(nxt=nxt, nb=1 - b):
                    start_in(nxt, nb)
        if i >= 2:
            wait_out(b)
        if i <= 8:
            compute(i, b)
        else:
            @pl.when(chunk_index(i) < NUM_CHUNKS)
            def _(i=i, b=b):
                compute(i, b)
    wait_out(0)

    @pl.when(chunk_index(CHUNKS_PER_WORKER - 1) < NUM_CHUNKS)
    def _():
        wait_out(1)


_sc_hist = functools.partial(
    pl.kernel,
    out_type=jax.ShapeDtypeStruct((3 * NUM_BINS, MP), jnp.float32),
    mesh=plsc.VectorSubcoreMesh(core_axis_name="c", subcore_axis_name="s"),
    compiler_params=pltpu.CompilerParams(needs_layout_passes=False),
    scratch_types=[
        pltpu.VMEM((2, N, CHUNK), jnp.float32),
        pltpu.VMEM((2, N, CHUNK), jnp.float32),
        pltpu.VMEM((2, CHUNK), jnp.int32),
        pltpu.VMEM((2, 3 * NUM_BINS, CHUNK), jnp.float32),
        pltpu.SemaphoreType.DMA,
        pltpu.SemaphoreType.DMA,
        pltpu.SemaphoreType.DMA,
        pltpu.SemaphoreType.DMA,
    ],
)(_sc_hist_body)


BM = 16384
G1 = pl.cdiv(MP, BM)


def _tc1_body(h_ref, wc_ref, wz_ref, wr_ref, b_ref,
              x_ref, st_ref, acc):
    i = pl.program_id(0)

    @pl.when(i == 0)
    def _():
        acc[...] = jnp.zeros_like(acc)

    cnt = h_ref[0:NUM_BINS, :]
    denom = cnt + 1e-5
    mz = h_ref[NUM_BINS:2 * NUM_BINS, :] / denom
    mr = h_ref[2 * NUM_BINS:3 * NUM_BINS, :] / denom
    x = (jnp.dot(wc_ref[...], cnt, preferred_element_type=jnp.float32)
         + jnp.dot(wz_ref[...], mz, preferred_element_type=jnp.float32)
         + jnp.dot(wr_ref[...], mr, preferred_element_type=jnp.float32)
         + b_ref[...])
    x_ref[...] = x
    mcol = i * BM + lax.broadcasted_iota(jnp.int32, (FEAT, BM), 1)
    xm = jnp.where(mcol < M, x, 0.0)
    acc[:, 0:1] += jnp.sum(xm, axis=1, keepdims=True)
    acc[:, 1:2] += jnp.sum(xm * xm, axis=1, keepdims=True)

    @pl.when(i == G1 - 1)
    def _():
        st_ref[...] = acc[...]


def _tc2_body(x_ref, st_ref, g_ref, be_ref, o_ref):
    s = st_ref[:, 0:1]
    ss = st_ref[:, 1:2]
    mu = s * (1.0 / M)
    var = jnp.maximum(ss * (1.0 / M) - mu * mu, 0.0)
    inv = 1.0 / jnp.sqrt(var + 1e-5)
    scale = g_ref[...] * inv
    shift = be_ref[...] - mu * scale
    o_ref[...] = jnp.maximum(x_ref[...] * scale + shift, 0.0)


def kernel(features, num_points, coors, W, b, gamma, beta):
    del coors
    zt = jnp.pad(features[:, :, 2].T, ((0, 0), (0, MP - M)))
    rt = jnp.pad(features[:, :, 3].T, ((0, 0), (0, MP - M)))
    np32 = jnp.pad(num_points.astype(jnp.int32), (0, MP - M))

    hist = _sc_hist(zt, rt, np32)

    wc = jnp.asarray(W[:, 0::3])
    wz = jnp.asarray(W[:, 1::3])
    wr = jnp.asarray(W[:, 2::3])

    xt, st = pl.pallas_call(
        _tc1_body,
        grid=(G1,),
        in_specs=[
            pl.BlockSpec((3 * NUM_BINS, BM), lambda i: (0, i)),
            pl.BlockSpec((FEAT, NUM_BINS), lambda i: (0, 0)),
            pl.BlockSpec((FEAT, NUM_BINS), lambda i: (0, 0)),
            pl.BlockSpec((FEAT, NUM_BINS), lambda i: (0, 0)),
            pl.BlockSpec((FEAT, 1), lambda i: (0, 0)),
        ],
        out_specs=[
            pl.BlockSpec((FEAT, BM), lambda i: (0, i)),
            pl.BlockSpec((FEAT, 128), lambda i: (0, 0)),
        ],
        out_shape=[
            jax.ShapeDtypeStruct((FEAT, M), jnp.float32),
            jax.ShapeDtypeStruct((FEAT, 128), jnp.float32),
        ],
        scratch_shapes=[pltpu.VMEM((FEAT, 128), jnp.float32)],
    )(hist, wc, wz, wr, b.reshape(FEAT, 1))

    out_t = pl.pallas_call(
        _tc2_body,
        grid=(G1,),
        in_specs=[
            pl.BlockSpec((FEAT, BM), lambda i: (0, i)),
            pl.BlockSpec((FEAT, 128), lambda i: (0, 0)),
            pl.BlockSpec((FEAT, 1), lambda i: (0, 0)),
            pl.BlockSpec((FEAT, 1), lambda i: (0, 0)),
        ],
        out_specs=pl.BlockSpec((FEAT, BM), lambda i: (0, i)),
        out_shape=jax.ShapeDtypeStruct((FEAT, M), jnp.float32),
    )(xt, st, gamma.reshape(FEAT, 1), beta.reshape(FEAT, 1))

    return out_t.T
